# Initial kernel scaffold; baseline (speedup 1.0000x reference)
import jax
import jax.numpy as jnp
from jax.experimental import pallas as pl


def kernel(x, y, params):
    raise NotImplementedError("write your pallas kernel here")


# speedup vs baseline: 1.4909x; 1.4909x over previous
import functools

import jax
import jax.numpy as jnp
from jax import lax
from jax.experimental import pallas as pl
from jax.experimental.pallas import tpu as pltpu
from jax.experimental.pallas import tpu_sc as plsc

B = 8
N = 2048
D = 64
NPOINT = 512
NSAMPLE = 32
BANDWIDTH = 0.1
EPS = 1e-5
P = B * NPOINT * NSAMPLE
GROUPS = B * NPOINT
TILE = 4096
GT = TILE // NSAMPLE


def _geom_body(x2_ref, xt_ref, newxyz_ref, gidx_ref, gnorm_ref, ds0_ref,
               invd_ref):
    b = pl.program_id(0)
    x2 = x2_ref[0]
    xt = xt_ref[0]
    n2 = jnp.sum(x2 * x2, axis=1)

    CH = 512
    for c in range(N // CH):
        rows = x2[c * CH:(c + 1) * CH]
        sq = -2.0 * jnp.dot(rows, xt, preferred_element_type=jnp.float32)
        sq = sq + n2[c * CH:(c + 1) * CH][:, None]
        sq = sq + n2[None, :]
        g = jnp.exp(-sq / (2.0 * BANDWIDTH * BANDWIDTH)) / (2.5 * BANDWIDTH)
        invd_ref[pl.ds(c * CH, CH), :] = (1.0 / jnp.mean(g, axis=1))[:, None]

    def fps_body(i, carry):
        dist_run, far, cvec = carry
        cvec = jnp.where(
            lax.broadcasted_iota(jnp.int32, (NPOINT,), 0) == i, far, cvec)
        cen = x2_ref[0, pl.ds(far, 1), :]
        d = jnp.sum((x2 - cen) ** 2, axis=1)
        dist_run = jnp.minimum(dist_run, d)
        mx = jnp.max(dist_run)
        far2 = jnp.min(jnp.where(
            dist_run == mx,
            lax.broadcasted_iota(jnp.int32, (N,), 0),
            jnp.int32(N))).astype(jnp.int32)
        return dist_run, far2, cvec

    dist0 = jnp.full((N,), 1e10, dtype=jnp.float32)
    _, _, cvec = lax.fori_loop(
        0, NPOINT, fps_body,
        (dist0, jnp.int32(0), jnp.zeros((NPOINT,), jnp.int32)))

    ohc = (cvec[:, None] ==
           lax.broadcasted_iota(jnp.int32, (NPOINT, N), 1)).astype(jnp.float32)
    newx = jnp.dot(ohc, x2, preferred_element_type=jnp.float32,
               precision=lax.Precision.HIGHEST)
    newxyz_ref[0] = newx

    nq = jnp.sum(newx * newx, axis=1)
    sqk = -2.0 * jnp.dot(newx, xt, preferred_element_type=jnp.float32)
    sqk = sqk + nq[:, None]
    sqk = sqk + n2[None, :]

    col = lax.broadcasted_iota(jnp.int32, (NPOINT, N), 1)
    invd = invd_ref[...]

    def knn_body(k, d):
        mn = jnp.min(d, axis=1, keepdims=True)
        m = jnp.min(jnp.where(d == mn, col, jnp.int32(N)),
                    axis=1).astype(jnp.int32)
        oh = col == m[:, None]
        gidx_ref[0, pl.ds(k, 1), :] = (m + b * N)[None, :]
        ohf = oh.astype(jnp.float32)
        g3 = jnp.dot(ohf, x2, preferred_element_type=jnp.float32,
               precision=lax.Precision.HIGHEST)
        g1 = jnp.dot(ohf, invd, preferred_element_type=jnp.float32,
               precision=lax.Precision.HIGHEST)
        for c in range(3):
            gnorm_ref[0, c, pl.ds(k, 1), :] = (g3[:, c] - newx[:, c])[None]
        ds0_ref[0, pl.ds(k, 1), :] = g1[:, 0][None]
        return jnp.where(oh, jnp.float32(jnp.inf), d)

    lax.fori_loop(0, NSAMPLE, knn_body, sqk)

    gd = ds0_ref[0]
    ds0_ref[0] = gd / jnp.max(gd, axis=0, keepdims=True)


_geom = pl.pallas_call(
    _geom_body,
    grid=(B,),
    in_specs=[pl.BlockSpec((1, N, 3), lambda b: (b, 0, 0)),
              pl.BlockSpec((1, 3, N), lambda b: (b, 0, 0))],
    out_specs=[pl.BlockSpec((1, NPOINT, 3), lambda b: (b, 0, 0)),
               pl.BlockSpec((1, NSAMPLE, NPOINT), lambda b: (b, 0, 0)),
               pl.BlockSpec((1, 3, NSAMPLE, NPOINT), lambda b: (b, 0, 0, 0)),
               pl.BlockSpec((1, NSAMPLE, NPOINT), lambda b: (b, 0, 0))],
    out_shape=[jax.ShapeDtypeStruct((B, NPOINT, 3), jnp.float32),
               jax.ShapeDtypeStruct((B, NSAMPLE, NPOINT), jnp.int32),
               jax.ShapeDtypeStruct((B, 3, NSAMPLE, NPOINT), jnp.float32),
               jax.ShapeDtypeStruct((B, NSAMPLE, NPOINT), jnp.float32)],
    scratch_shapes=[pltpu.VMEM((N, 1), jnp.float32)],
)


_NC = 2
_NS = 16
_NW = _NC * _NS
_CHUNK = 128
_ROWS_W = P // _NW
_CH_W = _ROWS_W // _CHUNK


def _sc_gather_body(tbl_ref, idx_ref, out_ref, idx_v, rows_v, sem):
    wid = lax.axis_index("s") * _NC + lax.axis_index("c")
    rbase = wid * _CH_W
    pltpu.sync_copy(idx_ref.at[pl.ds(rbase, _CH_W)], idx_v)

    def chunk(j, carry):
        pltpu.async_copy(tbl_ref.at[idx_v.at[j]], rows_v, sem).wait()
        pltpu.sync_copy(rows_v,
                        out_ref.at[pl.ds((rbase + j) * _CHUNK, _CHUNK)])
        return carry

    lax.fori_loop(0, _CH_W, chunk, 0)


@functools.cache
def _sc_gather_kernel():
    return pl.kernel(
        _sc_gather_body,
        mesh=plsc.VectorSubcoreMesh(core_axis_name="c",
                                    subcore_axis_name="s"),
        compiler_params=pltpu.CompilerParams(use_tc_tiling_on_sc=False),
        out_type=jax.ShapeDtypeStruct((P, D), jnp.float32),
        scratch_types=[pltpu.VMEM((_CH_W, _CHUNK), jnp.int32),
                       pltpu.VMEM((_CHUNK, D), jnp.float32),
                       pltpu.SemaphoreType.DMA],
    )


def _gather_feats(table, idx2d):
    return _sc_gather_kernel()(table, idx2d)


def _stats_update(o, s_ref, ss_ref):
    @pl.when(pl.program_id(0) == 0)
    def _():
        s_ref[...] = jnp.zeros_like(s_ref)
        ss_ref[...] = jnp.zeros_like(ss_ref)
    s_ref[...] += jnp.sum(o, axis=0, keepdims=True)
    ss_ref[...] += jnp.sum(o * o, axis=0, keepdims=True)


def _layer0_body(gn_ref, ft_ref, wa_ref, wb_ref, b_ref, out_ref, s_ref,
                 ss_ref):
    o = (jnp.dot(gn_ref[...], wa_ref[...], preferred_element_type=jnp.float32)
         + jnp.dot(ft_ref[...], wb_ref[...],
                   preferred_element_type=jnp.float32)
         + b_ref[...])
    out_ref[...] = o
    _stats_update(o, s_ref, ss_ref)


def _layer_body(act, pre_ref, a_ref, c_ref, w_ref, b_ref, out_ref, s_ref,
                ss_ref):
    x = act(pre_ref[...] * a_ref[...] + c_ref[...])
    o = (jnp.dot(x, w_ref[...], preferred_element_type=jnp.float32)
         + b_ref[...])
    out_ref[...] = o
    _stats_update(o, s_ref, ss_ref)


def _layer_call(body, ins, cins, cout, rows):
    nt = rows // TILE
    data_specs = []
    for a in ins:
        shp = a.shape
        if shp[0] == rows:
            data_specs.append(pl.BlockSpec((TILE, shp[1]),
                                           lambda i: (i, 0)))
        else:
            nd = len(shp)
            data_specs.append(pl.BlockSpec(shp,
                                           lambda i, _n=nd: (0,) * _n))
    return pl.pallas_call(
        body,
        grid=(nt,),
        in_specs=data_specs,
        out_specs=[pl.BlockSpec((TILE, cout), lambda i: (i, 0)),
                   pl.BlockSpec((1, cout), lambda i: (0, 0)),
                   pl.BlockSpec((1, cout), lambda i: (0, 0))],
        out_shape=[jax.ShapeDtypeStruct((rows, cout), jnp.float32),
                   jax.ShapeDtypeStruct((1, cout), jnp.float32),
                   jax.ShapeDtypeStruct((1, cout), jnp.float32)],
    )(*ins)


def _ac(s, ss, n, g, be):
    m = s[0] / n
    v = jnp.maximum(ss[0] / n - m * m, 0.0)
    a = g / jnp.sqrt(v + EPS)
    return a[None], (be - m * a)[None]


def _plane_chain_body(dims, last_sigmoid, x_ref, *refs):
    nl = len(dims) - 1
    prms = refs[:4 * nl]
    out_ref = refs[4 * nl]
    def b16(t):
        return t.astype(jnp.bfloat16).astype(jnp.float32)

    planes = [x_ref[c] for c in range(dims[0])]
    for li in range(nl):
        wt, bv, gv, bev = prms[4 * li:4 * li + 4]
        rounded = [b16(pp) for pp in planes]
        outs = []
        for o in range(dims[li + 1]):
            acc = rounded[0] * b16(wt[0, o])
            for ci in range(1, dims[li]):
                acc = acc + rounded[ci] * b16(wt[ci, o])
            acc = acc + bv[0, o]
            m = jnp.sum(acc) / float(P)
            v = jnp.sum((acc - m) ** 2) / float(P)
            xh = (acc - m) / jnp.sqrt(v + EPS) * gv[0, o] + bev[0, o]
            if last_sigmoid and li == nl - 1:
                outs.append(jax.nn.sigmoid(xh))
            else:
                outs.append(jnp.maximum(xh, 0.0))
        planes = outs
    for o in range(len(planes)):
        out_ref[o] = planes[o]


def _plane_chain(x, prms, dims, last_sigmoid):
    nl = len(dims) - 1
    smem = pl.BlockSpec(memory_space=pltpu.SMEM)
    return pl.pallas_call(
        functools.partial(_plane_chain_body, dims, last_sigmoid),
        in_specs=[pl.BlockSpec(memory_space=pltpu.VMEM)]
        + [smem] * (4 * nl),
        out_specs=pl.BlockSpec(memory_space=pltpu.VMEM),
        out_shape=jax.ShapeDtypeStruct((dims[-1], P // 128, 128),
                                       jnp.float32),
    )(x, *prms)


def _p2_body(pre2_ref, a2_ref, c2_ref, ds_ref, wn_ref, lwr_ref, lb_ref,
             out_ref, s_ref, ss_ref):
    h2 = jnp.maximum(pre2_ref[...] * a2_ref[...] + c2_ref[...], 0.0)
    h2 = h2 * ds_ref[...]
    h2 = h2.astype(jnp.bfloat16).astype(jnp.float32)
    wn = wn_ref[...].astype(jnp.bfloat16).astype(jnp.float32)
    acc = jnp.zeros((GT, 128), jnp.float32)
    for k in range(16):
        t = h2 * wn[:, k:k + 1]
        tg = jnp.sum(t.reshape(GT, NSAMPLE, 128), axis=1)
        acc = acc + jnp.dot(tg, lwr_ref[k],
                            preferred_element_type=jnp.float32)
    o = acc + lb_ref[...]
    out_ref[...] = o
    _stats_update(o, s_ref, ss_ref)


def _p3_body(pre_ref, a_ref, c_ref, out_ref):
    out_ref[...] = jnp.maximum(pre_ref[...] * a_ref[...] + c_ref[...], 0.0)


def kernel(x, y, params):
    p = params
    x2 = jnp.transpose(x, (0, 2, 1))
    newxyz, gidx, gnorm, ds0 = _geom(x2, x)

    gnr = jnp.transpose(gnorm, (0, 3, 2, 1)).reshape(P, 3)
    gnp = jnp.transpose(gnorm, (1, 0, 3, 2)).reshape(3, P // 128, 128)
    d0p = jnp.transpose(ds0, (0, 2, 1)).reshape(1, P // 128, 128)
    idx2d = jnp.transpose(gidx, (0, 2, 1)).reshape(P // 128, 128)
    table = jnp.transpose(y, (0, 2, 1)).reshape(B * N, D)

    feats = _gather_feats(table, idx2d)

    wprms = []
    for i in range(3):
        wprms += [p['ww%d' % i].T, p['wb%d' % i][None],
                  p['wg%d' % i][None], p['wbe%d' % i][None]]
    dprms = []
    for i in range(3):
        dprms += [p['dw%d' % i].T, p['db%d' % i][None],
                  p['dg%d' % i][None], p['dbe%d' % i][None]]
    wn_pl = _plane_chain(gnp, wprms, (3, 8, 8, 16), False)
    ds_pl = _plane_chain(d0p, dprms, (1, 16, 8, 1), True)
    wn = jnp.transpose(wn_pl.reshape(16, P), (1, 0))
    dsf = jnp.transpose(ds_pl.reshape(1, P), (1, 0))

    wa = p['mw0'][:, :3].T
    wb = p['mw0'][:, 3:].T
    pre1, s1, ss1 = _layer_call(
        _layer0_body, [gnr, feats, wa, wb, p['mb0'][None]], 67, 64, P)
    a1, c1 = _ac(s1, ss1, P, p['mg0'], p['mbe0'])

    pre2, s2, ss2 = _layer_call(
        functools.partial(_layer_body, lambda t: jnp.maximum(t, 0.0)),
        [pre1, a1, c1, p['mw1'].T, p['mb1'][None]], 64, 128, P)
    a2, c2 = _ac(s2, ss2, P, p['mg1'], p['mbe1'])

    lwr = jnp.transpose(p['lw'].reshape(128, 128, 16), (2, 1, 0))
    nt = GROUPS // GT
    preF, sF, ssF = pl.pallas_call(
        _p2_body,
        grid=(nt,),
        in_specs=[pl.BlockSpec((TILE, 128), lambda i: (i, 0)),
                  pl.BlockSpec((1, 128), lambda i: (0, 0)),
                  pl.BlockSpec((1, 128), lambda i: (0, 0)),
                  pl.BlockSpec((TILE, 1), lambda i: (i, 0)),
                  pl.BlockSpec((TILE, 16), lambda i: (i, 0)),
                  pl.BlockSpec((16, 128, 128), lambda i: (0, 0, 0)),
                  pl.BlockSpec((1, 128), lambda i: (0, 0))],
        out_specs=[pl.BlockSpec((GT, 128), lambda i: (i, 0)),
                   pl.BlockSpec((1, 128), lambda i: (0, 0)),
                   pl.BlockSpec((1, 128), lambda i: (0, 0))],
        out_shape=[jax.ShapeDtypeStruct((GROUPS, 128), jnp.float32),
                   jax.ShapeDtypeStruct((1, 128), jnp.float32),
                   jax.ShapeDtypeStruct((1, 128), jnp.float32)],
    )(pre2, a2, c2, dsf, wn, lwr, p['lb'][None])
    aF, cF = _ac(sF, ssF, GROUPS, p['lg'], p['lbe'])

    outF = pl.pallas_call(
        _p3_body,
        out_shape=jax.ShapeDtypeStruct((GROUPS, 128), jnp.float32),
    )(preF, aF, cF)

    out1 = jnp.transpose(newxyz, (0, 2, 1))
    out2 = jnp.transpose(outF.reshape(B, NPOINT, 128), (0, 2, 1))
    return out1, out2
